# Initial kernel scaffold; baseline (speedup 1.0000x reference)
import jax
import jax.numpy as jnp
from jax.experimental import pallas as pl


def kernel(x, nbrs, W):
    raise NotImplementedError("write your pallas kernel here")


# speedup vs baseline: 3.1592x; 3.1592x over previous
import functools
import math

import jax
import jax.numpy as jnp
from jax import lax
from jax.experimental import pallas as pl
from jax.experimental.pallas import tpu as pltpu
from jax.experimental.pallas import tpu_sc as plsc

N = 10000
DEG = 16
D = 256
TPERC = 0.45
REMOVE = math.floor(DEG * TPERC)
LO = REMOVE
HI = DEG - REMOVE - 1

NC = 2
NS = 16
L = 16
NW = NC * NS

NODES_PER_W = 320
NPAD = NW * NODES_PER_W
CH = 8
ROWS = CH * DEG
NCHUNK = NODES_PER_W // CH


def _mm_body(x_ref, wt_ref, o_ref):
    o_ref[...] = jnp.dot(x_ref[...], wt_ref[...],
                         preferred_element_type=jnp.float32)


def _project(x, wt):
    m = x.shape[0]
    blk = 1000
    grid = m // blk
    return pl.pallas_call(
        _mm_body,
        grid=(grid,),
        in_specs=[
            pl.BlockSpec((blk, D), lambda i: (i, 0)),
            pl.BlockSpec((D, D), lambda i: (0, 0)),
        ],
        out_specs=pl.BlockSpec((blk, D), lambda i: (i, 0)),
        out_shape=jax.ShapeDtypeStruct((m, D), jnp.float32),
    )(x, wt)


_sc_mesh = plsc.VectorSubcoreMesh(core_axis_name="c", subcore_axis_name="s")


@functools.partial(
    pl.kernel,
    mesh=_sc_mesh,
    out_type=jax.ShapeDtypeStruct((NPAD, D), jnp.float32),
    scratch_types=[
        pltpu.VMEM((ROWS,), jnp.int32),
        pltpu.VMEM((ROWS, D), jnp.float32),
        pltpu.VMEM((2 * D,), jnp.float32),
        pltpu.VMEM((CH, D), jnp.float32),
        pltpu.SemaphoreType.DMA,
    ],
    compiler_params=pltpu.CompilerParams(
        use_tc_tiling_on_sc=False, needs_layout_passes=False
    ),
)
def _sc_median(h_hbm, nbrs_hbm, out_hbm, idx_v, rows_v, pair_v, out_v, sem):
    cid = lax.axis_index("c")
    sid = lax.axis_index("s")
    wid = cid * NS + sid
    iota = lax.iota(jnp.int32, L)
    pair_mask = jnp.logical_or(iota == LO, iota == HI)
    pair_base = jnp.where(iota == HI, D, 0).astype(jnp.int32)

    def chunk_body(g, carry):
        node_base = wid * NODES_PER_W + g * CH
        pltpu.sync_copy(nbrs_hbm.at[pl.ds(node_base * DEG, ROWS)], idx_v)
        pltpu.async_copy(h_hbm.at[idx_v], rows_v, sem).wait()
        for n in range(CH):
            row_idx = n * DEG + iota

            def ch_body(c, carry2):
                col_idx = jnp.broadcast_to(c, (L,)).astype(jnp.int32)
                col = plsc.load_gather(rows_v, [row_idx, col_idx])
                s = jnp.sort(col)
                plsc.store_scatter(pair_v, [pair_base + c], s, mask=pair_mask)
                return carry2

            lax.fori_loop(0, D, ch_body, 0, unroll=8)
            for gg in range(D // L):
                lo = pair_v[pl.ds(gg * L, L)]
                hi = pair_v[pl.ds(D + gg * L, L)]
                out_v[n, pl.ds(gg * L, L)] = (lo + hi) * 0.5
        pltpu.sync_copy(out_v, out_hbm.at[pl.ds(node_base, CH)])
        return carry

    lax.fori_loop(0, NCHUNK, chunk_body, 0)


def kernel(x, nbrs, W):
    h = _project(x, W.T)
    nbrs_pad = jnp.zeros((NPAD, DEG), jnp.int32).at[:N].set(nbrs)
    out_pad = _sc_median(h, nbrs_pad.reshape(-1))
    return out_pad[:N]


# speedup vs baseline: 13.3866x; 4.2374x over previous
import functools
import math

import jax
import jax.numpy as jnp
from jax import lax
from jax.experimental import pallas as pl
from jax.experimental.pallas import tpu as pltpu
from jax.experimental.pallas import tpu_sc as plsc

N = 10000
DEG = 16
D = 256
TPERC = 0.45
REMOVE = math.floor(DEG * TPERC)
LO = REMOVE
HI = DEG - REMOVE - 1

NC = 2
NS = 16
L = 16
NW = NC * NS

NODES_PER_W = 320
NPAD = NW * NODES_PER_W
CH = 8
ROWS = CH * DEG
NCHUNK = NODES_PER_W // CH


def _mm_body(x_ref, wt_ref, o_ref):
    o_ref[...] = jnp.dot(x_ref[...], wt_ref[...],
                         preferred_element_type=jnp.float32)


def _project(x, wt):
    m = x.shape[0]
    blk = 1000
    grid = m // blk
    return pl.pallas_call(
        _mm_body,
        grid=(grid,),
        in_specs=[
            pl.BlockSpec((blk, D), lambda i: (i, 0)),
            pl.BlockSpec((D, D), lambda i: (0, 0)),
        ],
        out_specs=pl.BlockSpec((blk, D), lambda i: (i, 0)),
        out_shape=jax.ShapeDtypeStruct((m, D), jnp.float32),
    )(x, wt)


_SORT8 = ((0, 1), (2, 3), (4, 5), (6, 7), (0, 2), (1, 3), (4, 6), (5, 7),
          (1, 2), (5, 6), (0, 4), (1, 5), (2, 6), (3, 7), (2, 4), (3, 5),
          (1, 2), (3, 4), (5, 6))


def _median16(vs):
    vs = list(vs)
    for off in (0, 8):
        for (i, j) in _SORT8:
            a, b = vs[off + i], vs[off + j]
            vs[off + i] = jnp.minimum(a, b)
            vs[off + j] = jnp.maximum(a, b)
    lo = [jnp.minimum(vs[i], vs[15 - i]) for i in range(8)]
    hi = [jnp.maximum(vs[i], vs[15 - i]) for i in range(8)]
    while len(lo) > 1:
        lo = [jnp.maximum(lo[k], lo[k + 1]) for k in range(0, len(lo), 2)]
        hi = [jnp.minimum(hi[k], hi[k + 1]) for k in range(0, len(hi), 2)]
    return (lo[0] + hi[0]) * 0.5


_sc_mesh = plsc.VectorSubcoreMesh(core_axis_name="c", subcore_axis_name="s")


@functools.partial(
    pl.kernel,
    mesh=_sc_mesh,
    out_type=jax.ShapeDtypeStruct((NPAD, D), jnp.float32),
    scratch_types=[
        pltpu.VMEM((ROWS,), jnp.int32),
        pltpu.VMEM((ROWS, D), jnp.float32),
        pltpu.VMEM((CH, D), jnp.float32),
        pltpu.SemaphoreType.DMA,
    ],
    compiler_params=pltpu.CompilerParams(
        use_tc_tiling_on_sc=False, needs_layout_passes=False
    ),
)
def _sc_median(h_hbm, nbrs_hbm, out_hbm, idx_v, rows_v, out_v, sem):
    cid = lax.axis_index("c")
    sid = lax.axis_index("s")
    wid = cid * NS + sid

    def chunk_body(g, carry):
        node_base = wid * NODES_PER_W + g * CH
        pltpu.sync_copy(nbrs_hbm.at[pl.ds(node_base * DEG, ROWS)], idx_v)
        pltpu.async_copy(h_hbm.at[idx_v], rows_v, sem).wait()
        for n in range(CH):

            def grp_body(gg, carry2):
                cs = pl.ds(gg * L, L)
                vs = [rows_v[n * DEG + j, cs] for j in range(DEG)]
                out_v[n, cs] = _median16(vs)
                return carry2

            lax.fori_loop(0, D // L, grp_body, 0, unroll=2)
        pltpu.sync_copy(out_v, out_hbm.at[pl.ds(node_base, CH)])
        return carry

    lax.fori_loop(0, NCHUNK, chunk_body, 0)


def kernel(x, nbrs, W):
    h = _project(x, W.T)
    nbrs_pad = jnp.zeros((NPAD, DEG), jnp.int32).at[:N].set(nbrs)
    out_pad = _sc_median(h, nbrs_pad.reshape(-1))
    return out_pad[:N]


# speedup vs baseline: 25.1523x; 1.8789x over previous
import functools
import math

import jax
import jax.numpy as jnp
from jax import lax
from jax.experimental import pallas as pl
from jax.experimental.pallas import tpu as pltpu
from jax.experimental.pallas import tpu_sc as plsc

N = 10000
DEG = 16
D = 256
TPERC = 0.45
REMOVE = math.floor(DEG * TPERC)
LO = REMOVE
HI = DEG - REMOVE - 1

NC = 2
NS = 16
L = 16
NW = NC * NS

CH = 8
ROWS = CH * DEG
NCHUNKS = N // CH
KMAX = -(-NCHUNKS // NW)


def _mm_body(x_ref, wt_ref, o_ref):
    o_ref[...] = jnp.dot(x_ref[...], wt_ref[...],
                         preferred_element_type=jnp.float32)


def _project(x, wt):
    m = x.shape[0]
    blk = 1000
    grid = m // blk
    return pl.pallas_call(
        _mm_body,
        grid=(grid,),
        in_specs=[
            pl.BlockSpec((blk, D), lambda i: (i, 0)),
            pl.BlockSpec((D, D), lambda i: (0, 0)),
        ],
        out_specs=pl.BlockSpec((blk, D), lambda i: (i, 0)),
        out_shape=jax.ShapeDtypeStruct((m, D), jnp.float32),
    )(x, wt)


_SORT8 = ((0, 1), (2, 3), (4, 5), (6, 7), (0, 2), (1, 3), (4, 6), (5, 7),
          (1, 2), (5, 6), (0, 4), (1, 5), (2, 6), (3, 7), (2, 4), (3, 5),
          (1, 2), (3, 4), (5, 6))


def _median16(vs):
    vs = list(vs)
    for off in (0, 8):
        for (i, j) in _SORT8:
            a, b = vs[off + i], vs[off + j]
            vs[off + i] = jnp.minimum(a, b)
            vs[off + j] = jnp.maximum(a, b)
    lo = [jnp.minimum(vs[i], vs[15 - i]) for i in range(8)]
    hi = [jnp.maximum(vs[i], vs[15 - i]) for i in range(8)]
    while len(lo) > 1:
        lo = [jnp.maximum(lo[k], lo[k + 1]) for k in range(0, len(lo), 2)]
        hi = [jnp.minimum(hi[k], hi[k + 1]) for k in range(0, len(hi), 2)]
    return (lo[0] + hi[0]) * 0.5


_sc_mesh = plsc.VectorSubcoreMesh(core_axis_name="c", subcore_axis_name="s")


@functools.partial(
    pl.kernel,
    mesh=_sc_mesh,
    out_type=jax.ShapeDtypeStruct((N, D), jnp.float32),
    scratch_types=[
        pltpu.VMEM((ROWS,), jnp.int32),
        pltpu.VMEM((ROWS,), jnp.int32),
        pltpu.VMEM((ROWS, D), jnp.float32),
        pltpu.VMEM((ROWS, D), jnp.float32),
        pltpu.VMEM((CH, D), jnp.float32),
        pltpu.SemaphoreType.DMA,
        pltpu.SemaphoreType.DMA,
    ],
    compiler_params=pltpu.CompilerParams(
        use_tc_tiling_on_sc=False, needs_layout_passes=False
    ),
)
def _sc_median(h_hbm, nbrs_hbm, out_hbm, idx0, idx1, rows0, rows1, out_v,
               sem0, sem1):
    cid = lax.axis_index("c")
    sid = lax.axis_index("s")
    wid = cid * NS + sid
    idx_b = (idx0, idx1)
    rows_b = (rows0, rows1)
    sem_b = (sem0, sem1)

    def start(k, b):
        chunk = wid + NW * k

        @pl.when(chunk < NCHUNKS)
        def _():
            pltpu.sync_copy(nbrs_hbm.at[pl.ds(chunk * ROWS, ROWS)], idx_b[b])
            pltpu.async_copy(h_hbm.at[idx_b[b]], rows_b[b], sem_b[b])

    def finish(k, b):
        chunk = wid + NW * k

        @pl.when(chunk < NCHUNKS)
        def _():
            pltpu.make_async_copy(h_hbm.at[idx_b[b]], rows_b[b],
                                  sem_b[b]).wait()
            rows_v = rows_b[b]
            for n in range(CH):

                def grp_body(gg, carry2):
                    cs = pl.ds(gg * L, L)
                    vs = [rows_v[n * DEG + j, cs] for j in range(DEG)]
                    out_v[n, cs] = _median16(vs)
                    return carry2

                lax.fori_loop(0, D // L, grp_body, 0, unroll=2)
            pltpu.sync_copy(out_v, out_hbm.at[pl.ds(chunk * CH, CH)])

    start(0, 0)

    def pair_body(k2, carry):
        k = 2 * k2
        start(k + 1, 1)
        finish(k, 0)
        start(k + 2, 0)
        finish(k + 1, 1)
        return carry

    lax.fori_loop(0, KMAX // 2, pair_body, 0)


def kernel(x, nbrs, W):
    h = _project(x, W.T)
    return _sc_median(h, nbrs.reshape(-1))
